# baseline (speedup 1.0000x reference)
import jax
import jax.numpy as jnp
from jax import lax
from jax.experimental import pallas as pl
from jax.experimental.pallas import tpu as pltpu
from jax.experimental.pallas import tpu_sc as plsc

N_NODES = 10000
E_EDGES = 320000
D_IN = 128
U_HID = 16

NC = 2
NS = 16
N_TILES = NC * NS
CHUNK = 128
CHUNKS_PER_TILE = 80
EDGES_PER_TILE = CHUNKS_PER_TILE * CHUNK
E_PAD = N_TILES * EDGES_PER_TILE
N_PAD = 10240
ROWS_PER_TILE = N_PAD // NS
NBUF = 4
CHUNKS_SRC = CHUNKS_PER_TILE + NBUF


def _deg_body(dst_hbm, ones_hbm, zeros_hbm, out_hbm, acc, didx, ones_v, sem):
    cid = lax.axis_index("c")
    sid = lax.axis_index("s")
    row0 = sid * ROWS_PER_TILE
    rows = pl.ds(row0, ROWS_PER_TILE)
    pltpu.sync_copy(zeros_hbm.at[rows], acc.at[rows])
    pltpu.sync_copy(ones_hbm, ones_v)
    tile = cid * NS + sid
    pltpu.sync_copy(dst_hbm.at[tile], didx)
    plsc.subcore_barrier()

    @pl.loop(0, CHUNKS_PER_TILE)
    def _(k):
        pltpu.async_copy(ones_v, acc.at[didx.at[k]], sem, add=True)

    @pl.loop(0, CHUNKS_PER_TILE)
    def _(k):
        pltpu.make_async_copy(ones_hbm, ones_v, sem).wait()

    plsc.subcore_barrier()
    pltpu.sync_copy(acc.at[rows], out_hbm.at[cid].at[rows])


def _scat_body(g_hbm, src_hbm, dst_hbm, zeros_hbm, out_hbm,
               acc, sidx, didx, bufs, sems):
    cid = lax.axis_index("c")
    sid = lax.axis_index("s")
    row0 = sid * ROWS_PER_TILE
    rows = pl.ds(row0, ROWS_PER_TILE)
    pltpu.sync_copy(zeros_hbm.at[rows], acc.at[rows])
    tile = cid * NS + sid
    pltpu.sync_copy(src_hbm.at[tile], sidx)
    pltpu.sync_copy(dst_hbm.at[tile], didx)
    plsc.subcore_barrier()

    for b in range(NBUF):
        pltpu.async_copy(g_hbm.at[sidx.at[b]], bufs.at[b], sems.at[b])

    @pl.loop(0, CHUNKS_PER_TILE, step=NBUF)
    def _(k):
        for b in range(NBUF):
            c = k + b
            pltpu.make_async_copy(g_hbm.at[sidx.at[b]], bufs.at[b],
                                  sems.at[b]).wait()
            pltpu.sync_copy(bufs.at[b], acc.at[didx.at[c]], add=True)
            pltpu.async_copy(g_hbm.at[sidx.at[c + NBUF]], bufs.at[b],
                             sems.at[b])

    for b in range(NBUF):
        pltpu.make_async_copy(g_hbm.at[sidx.at[b]], bufs.at[b],
                              sems.at[b]).wait()

    plsc.subcore_barrier()
    pltpu.sync_copy(acc.at[rows], out_hbm.at[cid].at[rows])


_sc_calls_cache = []


def _sc_calls():
    if not _sc_calls_cache:
        mesh = plsc.VectorSubcoreMesh(core_axis_name="c", subcore_axis_name="s",
                                      num_cores=NC, num_subcores=NS)
        cp = pltpu.CompilerParams(use_tc_tiling_on_sc=False)
        deg_call = pl.kernel(
            _deg_body,
            out_type=jax.ShapeDtypeStruct((NC, N_PAD, U_HID), jnp.float32),
            mesh=mesh,
            scratch_types=[
                pltpu.VMEM_SHARED((N_PAD, U_HID), jnp.float32),
                pltpu.VMEM((CHUNKS_PER_TILE, CHUNK), jnp.int32),
                pltpu.VMEM((CHUNK, U_HID), jnp.float32),
                pltpu.SemaphoreType.DMA,
            ],
            compiler_params=cp,
        )
        scat_call = pl.kernel(
            _scat_body,
            out_type=jax.ShapeDtypeStruct((NC, N_PAD, U_HID), jnp.float32),
            mesh=mesh,
            scratch_types=[
                pltpu.VMEM_SHARED((N_PAD, U_HID), jnp.float32),
                pltpu.VMEM((CHUNKS_SRC, CHUNK), jnp.int32),
                pltpu.VMEM((CHUNKS_PER_TILE, CHUNK), jnp.int32),
                pltpu.VMEM((NBUF, CHUNK, U_HID), jnp.float32),
                pltpu.SemaphoreType.DMA((NBUF,)),
            ],
            compiler_params=cp,
        )
        _sc_calls_cache.append((deg_call, scat_call))
    return _sc_calls_cache[0]


def _mm1_body(x_ref, w_ref, o_ref):
    o_ref[...] = jnp.dot(x_ref[...], w_ref[...],
                         preferred_element_type=jnp.float32)


def _combine1_body(p_ref, hw_ref, dinv_ref, g_ref):
    deg = p_ref[0] + p_ref[1] + 1.0
    dinv = lax.rsqrt(jnp.maximum(deg, 1e-12))
    dinv_ref[...] = dinv
    g_ref[...] = dinv * hw_ref[...]


def _layer2_body(p_ref, g1_ref, dinv_ref, w2_ref, b1_ref, g2_ref):
    dinv = dinv_ref[...]
    h1 = jnp.maximum(dinv * (p_ref[0] + p_ref[1] + g1_ref[...]) + b1_ref[...],
                     0.0)
    hw2 = jnp.dot(h1, w2_ref[...], preferred_element_type=jnp.float32)
    g2_ref[...] = dinv * hw2


def _final_body(p_ref, g2_ref, dinv_ref, b2_ref, o_ref):
    o_ref[...] = jnp.maximum(
        dinv_ref[...] * (p_ref[0] + p_ref[1] + g2_ref[...]) + b2_ref[...], 0.0)


def kernel(x, edge_index, W1, b1, W2, b2):
    src = edge_index[0]
    dst = edge_index[1]
    pad_e = E_PAD - E_EDGES
    pad_idx = jnp.full((pad_e,), N_NODES, dtype=jnp.int32)
    srcp = jnp.concatenate([src, pad_idx]).reshape(
        N_TILES, CHUNKS_PER_TILE, CHUNK)
    srcp = jnp.concatenate(
        [srcp, jnp.full((N_TILES, NBUF, CHUNK), N_NODES, jnp.int32)], axis=1)
    dstp = jnp.concatenate([dst, pad_idx]).reshape(
        N_TILES, CHUNKS_PER_TILE, CHUNK)
    x_pad = jnp.pad(x, ((0, N_PAD - N_NODES), (0, 0)))
    zeros = jnp.zeros((N_PAD, U_HID), jnp.float32)
    ones128 = jnp.ones((CHUNK, U_HID), jnp.float32)
    b1r = b1.reshape(1, U_HID)
    b2r = b2.reshape(1, U_HID)

    f32 = jnp.float32
    nu = jax.ShapeDtypeStruct((N_PAD, U_HID), f32)
    _deg_call, _scat_call = _sc_calls()

    deg_part = _deg_call(dstp, ones128, zeros)
    hw1 = pl.pallas_call(_mm1_body, out_shape=nu)(x_pad, W1)
    dinv, g1 = pl.pallas_call(_combine1_body, out_shape=(nu, nu))(
        deg_part, hw1)
    part1 = _scat_call(g1, srcp, dstp, zeros)
    g2 = pl.pallas_call(_layer2_body, out_shape=nu)(
        part1, g1, dinv, W2, b1r)
    part2 = _scat_call(g2, srcp, dstp, zeros)
    out = pl.pallas_call(_final_body, out_shape=nu)(part2, g2, dinv, b2r)
    return out[:N_NODES]


# speedup vs baseline: 2.2102x; 2.2102x over previous
import jax
import jax.numpy as jnp
from jax import lax
from jax.experimental import pallas as pl
from jax.experimental.pallas import tpu as pltpu
from jax.experimental.pallas import tpu_sc as plsc

N_NODES = 10000
E_EDGES = 320000
D_IN = 128
U_HID = 16

NC = 2
NS = 16
N_TILES = NC * NS
CHUNK = 128
CHUNKS_PER_TILE = 80
EDGES_PER_TILE = CHUNKS_PER_TILE * CHUNK
E_PAD = N_TILES * EDGES_PER_TILE
N_PAD = 10240
ROWS_PER_TILE = N_PAD // NS
NBUF = 4
CHUNKS_SRC = CHUNKS_PER_TILE + NBUF


def _deg_body(dst_hbm, ones_hbm, zeros_hbm, out_hbm, acc, didx, ones_v, sem):
    cid = lax.axis_index("c")
    sid = lax.axis_index("s")
    row0 = sid * ROWS_PER_TILE
    rows = pl.ds(row0, ROWS_PER_TILE)
    pltpu.sync_copy(zeros_hbm.at[rows], acc.at[rows])
    pltpu.sync_copy(ones_hbm, ones_v)
    tile = cid * NS + sid
    pltpu.sync_copy(dst_hbm.at[tile], didx)
    plsc.subcore_barrier()

    @pl.loop(0, CHUNKS_PER_TILE)
    def _(k):
        pltpu.async_copy(ones_v, acc.at[didx.at[k]], sem, add=True)

    @pl.loop(0, CHUNKS_PER_TILE)
    def _(k):
        pltpu.make_async_copy(ones_hbm, ones_v, sem).wait()

    plsc.subcore_barrier()
    pltpu.sync_copy(acc.at[rows], out_hbm.at[cid].at[rows])


def _scat_body(g_hbm, src_hbm, dst_hbm, zeros_hbm, out_hbm,
               acc, gv, sidx, didx, bufs, sems):
    cid = lax.axis_index("c")
    sid = lax.axis_index("s")
    row0 = sid * ROWS_PER_TILE
    rows = pl.ds(row0, ROWS_PER_TILE)
    pltpu.sync_copy(zeros_hbm.at[rows], acc.at[rows])
    pltpu.sync_copy(g_hbm.at[rows], gv.at[rows])
    tile = cid * NS + sid
    pltpu.sync_copy(src_hbm.at[tile], sidx)
    pltpu.sync_copy(dst_hbm.at[tile], didx)
    plsc.subcore_barrier()

    for b in range(NBUF):
        pltpu.async_copy(gv.at[sidx.at[b]], bufs.at[b], sems.at[b])

    @pl.loop(0, CHUNKS_PER_TILE, step=NBUF)
    def _(k):
        for b in range(NBUF):
            c = k + b
            pltpu.make_async_copy(gv.at[sidx.at[b]], bufs.at[b],
                                  sems.at[b]).wait()
            pltpu.sync_copy(bufs.at[b], acc.at[didx.at[c]], add=True)
            pltpu.async_copy(gv.at[sidx.at[c + NBUF]], bufs.at[b],
                             sems.at[b])

    for b in range(NBUF):
        pltpu.make_async_copy(gv.at[sidx.at[b]], bufs.at[b],
                              sems.at[b]).wait()

    plsc.subcore_barrier()
    pltpu.sync_copy(acc.at[rows], out_hbm.at[cid].at[rows])


_sc_calls_cache = []


def _sc_calls():
    if not _sc_calls_cache:
        mesh = plsc.VectorSubcoreMesh(core_axis_name="c", subcore_axis_name="s",
                                      num_cores=NC, num_subcores=NS)
        cp = pltpu.CompilerParams(use_tc_tiling_on_sc=False)
        deg_call = pl.kernel(
            _deg_body,
            out_type=jax.ShapeDtypeStruct((NC, N_PAD, U_HID), jnp.float32),
            mesh=mesh,
            scratch_types=[
                pltpu.VMEM_SHARED((N_PAD, U_HID), jnp.float32),
                pltpu.VMEM((CHUNKS_PER_TILE, CHUNK), jnp.int32),
                pltpu.VMEM((CHUNK, U_HID), jnp.float32),
                pltpu.SemaphoreType.DMA,
            ],
            compiler_params=cp,
        )
        scat_call = pl.kernel(
            _scat_body,
            out_type=jax.ShapeDtypeStruct((NC, N_PAD, U_HID), jnp.float32),
            mesh=mesh,
            scratch_types=[
                pltpu.VMEM_SHARED((N_PAD, U_HID), jnp.float32),
                pltpu.VMEM_SHARED((N_PAD, U_HID), jnp.float32),
                pltpu.VMEM((CHUNKS_SRC, CHUNK), jnp.int32),
                pltpu.VMEM((CHUNKS_PER_TILE, CHUNK), jnp.int32),
                pltpu.VMEM((NBUF, CHUNK, U_HID), jnp.float32),
                pltpu.SemaphoreType.DMA((NBUF,)),
            ],
            compiler_params=cp,
        )
        _sc_calls_cache.append((deg_call, scat_call))
    return _sc_calls_cache[0]


def _mm1_body(x_ref, w_ref, o_ref):
    o_ref[...] = jnp.dot(x_ref[...], w_ref[...],
                         preferred_element_type=jnp.float32)


def _combine1_body(p_ref, hw_ref, dinv_ref, g_ref):
    deg = p_ref[0] + p_ref[1] + 1.0
    dinv = lax.rsqrt(jnp.maximum(deg, 1e-12))
    dinv_ref[...] = dinv
    g_ref[...] = dinv * hw_ref[...]


def _layer2_body(p_ref, g1_ref, dinv_ref, w2_ref, b1_ref, g2_ref):
    dinv = dinv_ref[...]
    h1 = jnp.maximum(dinv * (p_ref[0] + p_ref[1] + g1_ref[...]) + b1_ref[...],
                     0.0)
    hw2 = jnp.dot(h1, w2_ref[...], preferred_element_type=jnp.float32)
    g2_ref[...] = dinv * hw2


def _final_body(p_ref, g2_ref, dinv_ref, b2_ref, o_ref):
    o_ref[...] = jnp.maximum(
        dinv_ref[...] * (p_ref[0] + p_ref[1] + g2_ref[...]) + b2_ref[...], 0.0)


def kernel(x, edge_index, W1, b1, W2, b2):
    src = edge_index[0]
    dst = edge_index[1]
    pad_e = E_PAD - E_EDGES
    pad_idx = jnp.full((pad_e,), N_NODES, dtype=jnp.int32)
    srcp = jnp.concatenate([src, pad_idx]).reshape(
        N_TILES, CHUNKS_PER_TILE, CHUNK)
    srcp = jnp.concatenate(
        [srcp, jnp.full((N_TILES, NBUF, CHUNK), N_NODES, jnp.int32)], axis=1)
    dstp = jnp.concatenate([dst, pad_idx]).reshape(
        N_TILES, CHUNKS_PER_TILE, CHUNK)
    x_pad = jnp.pad(x, ((0, N_PAD - N_NODES), (0, 0)))
    zeros = jnp.zeros((N_PAD, U_HID), jnp.float32)
    ones128 = jnp.ones((CHUNK, U_HID), jnp.float32)
    b1r = b1.reshape(1, U_HID)
    b2r = b2.reshape(1, U_HID)

    f32 = jnp.float32
    nu = jax.ShapeDtypeStruct((N_PAD, U_HID), f32)
    _deg_call, _scat_call = _sc_calls()

    deg_part = _deg_call(dstp, ones128, zeros)
    hw1 = pl.pallas_call(_mm1_body, out_shape=nu)(x_pad, W1)
    dinv, g1 = pl.pallas_call(_combine1_body, out_shape=(nu, nu))(
        deg_part, hw1)
    part1 = _scat_call(g1, srcp, dstp, zeros)
    g2 = pl.pallas_call(_layer2_body, out_shape=nu)(
        part1, g1, dinv, W2, b1r)
    part2 = _scat_call(g2, srcp, dstp, zeros)
    out = pl.pallas_call(_final_body, out_shape=nu)(part2, g2, dinv, b2r)
    return out[:N_NODES]
